# speedup vs baseline: 2.8548x; 2.8548x over previous
import jax
import jax.numpy as jnp
from jax.experimental import pallas as pl


def _add_kernel(x_ref, p_ref, o_ref):
    o_ref[...] = x_ref[...] + p_ref[...]


def kernel(x, pos_table):
    B, S, D = x.shape
    BS = 512
    return pl.pallas_call(
        _add_kernel,
        grid=(S // BS, B),
        in_specs=[
            pl.BlockSpec((1, BS, D), lambda j, b: (b, j, 0)),
            pl.BlockSpec((BS, D), lambda j, b: (j, 0)),
        ],
        out_specs=pl.BlockSpec((1, BS, D), lambda j, b: (b, j, 0)),
        out_shape=jax.ShapeDtypeStruct((B, S, D), x.dtype),
    )(x, pos_table)
